# baseline (speedup 1.0000x reference)
import functools

import jax
import jax.numpy as jnp
from jax import lax
from jax.experimental import pallas as pl
from jax.experimental.pallas import tpu as pltpu
from jax.experimental.pallas import tpu_sc as plsc

N = 10000
D = 128
DH = D // 2
E = 320000

NPAD = 10240
PAD_DST = 10008
NCORES = 2
NSUB = 16
CHUNK = 128
NCHUNK = 160
BCHUNKS = 10
EPT = CHUNK * NCHUNK
EPAD = EPT * NSUB
RPT = NPAD // NSUB
RCHUNKS = RPT // CHUNK


def _sc_body(x0_hbm, x1_hbm, idx_hbm, zrow_hbm, zdeg_hbm, ones_hbm,
             agg_out, deg_out, ib_v, rows_a, rows_b, rows_c, rows_d, rows_e,
             rows_f, ones_v, zbuf_v, zdbuf_v, acc_sh, deg_sh,
             sem_g0, sem_g1, sem_g2, sem_s0, sem_s1, sem_d0, sem_d1):
    c = lax.axis_index("c")
    s = lax.axis_index("s")
    r0 = s * RPT

    pltpu.sync_copy(zrow_hbm, zbuf_v)
    pltpu.sync_copy(zdeg_hbm, zdbuf_v)
    pltpu.sync_copy(ones_hbm, ones_v)
    for k in range(RCHUNKS):
        pltpu.sync_copy(zbuf_v, acc_sh.at[pl.ds(r0 + k * CHUNK, CHUNK)])
        pltpu.sync_copy(zdbuf_v, deg_sh.at[pl.ds(r0 + k * CHUNK, CHUNK)])
    plsc.subcore_barrier()

    rows = (rows_a, rows_b, rows_c, rows_d, rows_e, rows_f)
    sem_g = (sem_g0, sem_g1, sem_g2)
    sem_s = (sem_s0, sem_s1)
    sem_d = (sem_d0, sem_d1)
    ibase = s * (NCHUNK * 2)

    @pl.loop(0, NCHUNK // BCHUNKS)
    def _(bi):
        pltpu.sync_copy(idx_hbm.at[pl.ds(ibase + bi * (2 * BCHUNKS),
                                         2 * BCHUNKS)], ib_v)

        def issue_gather(j):
            r = rows[j % 6]
            g = sem_g[j % 3]

            @pl.when(c == 0)
            def _():
                pltpu.async_copy(x0_hbm.at[ib_v.at[2 * j]], r, g)

            @pl.when(c == 1)
            def _():
                pltpu.async_copy(x1_hbm.at[ib_v.at[2 * j]], r, g)

        def wait_gather(j):
            pltpu.make_async_copy(x0_hbm.at[ib_v.at[2 * j]], rows[j % 6],
                                  sem_g[j % 3]).wait()

        def issue_scatter(j):
            pltpu.async_copy(rows[j % 6], acc_sh.at[ib_v.at[2 * j + 1]],
                             sem_s[j % 2], add=True)

            @pl.when(c == 1)
            def _():
                pltpu.async_copy(ones_v, deg_sh.at[ib_v.at[2 * j + 1]],
                                 sem_d[j % 2], add=True)

        def wait_scatter(j):
            pltpu.make_async_copy(rows[j % 6], acc_sh.at[ib_v.at[2 * j + 1]],
                                  sem_s[j % 2]).wait()

            @pl.when(c == 1)
            def _():
                pltpu.make_async_copy(ones_v, deg_sh.at[ib_v.at[2 * j + 1]],
                                      sem_d[j % 2]).wait()

        issue_gather(0)
        issue_gather(1)
        issue_gather(2)
        for j in range(BCHUNKS):
            wait_gather(j)
            if j >= 2:
                wait_scatter(j - 2)
            issue_scatter(j)
            if j + 3 < BCHUNKS:
                issue_gather(j + 3)
        wait_scatter(BCHUNKS - 2)
        wait_scatter(BCHUNKS - 1)

    plsc.subcore_barrier()
    out0 = c * NPAD + r0
    for k in range(3):
        pltpu.async_copy(acc_sh.at[pl.ds(r0 + k * CHUNK, CHUNK)],
                         rows[k % 3], sem_g[k % 3])
    for k in range(RCHUNKS):
        pltpu.make_async_copy(acc_sh.at[pl.ds(r0 + k * CHUNK, CHUNK)],
                              rows[k % 3], sem_g[k % 3]).wait()
        pltpu.sync_copy(rows[k % 3],
                        agg_out.at[pl.ds(out0 + k * CHUNK, CHUNK)])
        if k + 3 < RCHUNKS:
            pltpu.async_copy(acc_sh.at[pl.ds(r0 + (k + 3) * CHUNK, CHUNK)],
                             rows[k % 3], sem_g[k % 3])
        pltpu.sync_copy(deg_sh.at[pl.ds(r0 + k * CHUNK, CHUNK)], zdbuf_v)
        pltpu.sync_copy(zdbuf_v, deg_out.at[pl.ds(out0 + k * CHUNK, CHUNK)])


@functools.cache
def _sc_pass_kernel():
    mesh = plsc.VectorSubcoreMesh(core_axis_name="c", subcore_axis_name="s")
    return pl.kernel(
        _sc_body,
        compiler_params=pltpu.CompilerParams(use_tc_tiling_on_sc=False),
        out_type=[
            jax.ShapeDtypeStruct((NCORES * NPAD, DH), jnp.float32),
            jax.ShapeDtypeStruct((NCORES * NPAD, 16), jnp.float32),
        ],
        mesh=mesh,
        scratch_types=[
            pltpu.VMEM((2 * BCHUNKS, CHUNK), jnp.int32),
            pltpu.VMEM((CHUNK, DH), jnp.float32),
            pltpu.VMEM((CHUNK, DH), jnp.float32),
            pltpu.VMEM((CHUNK, DH), jnp.float32),
            pltpu.VMEM((CHUNK, DH), jnp.float32),
            pltpu.VMEM((CHUNK, DH), jnp.float32),
            pltpu.VMEM((CHUNK, DH), jnp.float32),
            pltpu.VMEM((CHUNK, 16), jnp.float32),
            pltpu.VMEM((CHUNK, DH), jnp.float32),
            pltpu.VMEM((CHUNK, 16), jnp.float32),
            pltpu.VMEM_SHARED((NPAD, DH), jnp.float32),
            pltpu.VMEM_SHARED((NPAD, 16), jnp.float32),
            pltpu.SemaphoreType.DMA,
            pltpu.SemaphoreType.DMA,
            pltpu.SemaphoreType.DMA,
            pltpu.SemaphoreType.DMA,
            pltpu.SemaphoreType.DMA,
            pltpu.SemaphoreType.DMA,
            pltpu.SemaphoreType.DMA,
        ],
    )


def _sc_pass(x0, x1, idx_all, zrow, zdeg, ones):
    aggp, degp = _sc_pass_kernel()(x0, x1, idx_all, zrow, zdeg, ones)
    return aggp.reshape(NCORES, NPAD, DH), degp.reshape(NCORES, NPAD, 16)


ROWB = 2000


def _tc_matmul_body(x_ref, w_ref, b_ref, o_ref):
    o_ref[...] = lax.dot_general(
        x_ref[...], w_ref[...], (((1,), (1,)), ((), ())),
        preferred_element_type=jnp.float32) + b_ref[...]


def _tc_matmul(xin, W, b):
    return pl.pallas_call(
        _tc_matmul_body,
        grid=(N // ROWB,),
        in_specs=[
            pl.BlockSpec((ROWB, D), lambda i: (i, 0)),
            pl.BlockSpec((D, D), lambda i: (0, 0)),
            pl.BlockSpec((1, D), lambda i: (0, 0)),
        ],
        out_specs=pl.BlockSpec((ROWB, D), lambda i: (i, 0)),
        out_shape=jax.ShapeDtypeStruct((N, D), jnp.float32),
    )(xin, W, b)


def _tc_combine_body(relu, agg_ref, deg_ref, xr_ref, wl_ref, o_ref):
    a = jnp.concatenate([agg_ref[0], agg_ref[1]], axis=1)
    dg = deg_ref[0][:, 0:1] + deg_ref[1][:, 0:1]
    a = a / jnp.maximum(dg, 1.0)
    acc = lax.dot_general(a, wl_ref[...], (((1,), (1,)), ((), ())),
                          preferred_element_type=jnp.float32)
    acc = acc + xr_ref[...]
    if relu:
        acc = jnp.maximum(acc, 0.0)
    o_ref[...] = acc


def _tc_combine(aggp, degp, xr, W_l, relu):
    return pl.pallas_call(
        functools.partial(_tc_combine_body, relu),
        grid=(N // ROWB,),
        in_specs=[
            pl.BlockSpec((NCORES, ROWB, DH), lambda i: (0, i, 0)),
            pl.BlockSpec((NCORES, ROWB, 16), lambda i: (0, i, 0)),
            pl.BlockSpec((ROWB, D), lambda i: (i, 0)),
            pl.BlockSpec((D, D), lambda i: (0, 0)),
        ],
        out_specs=pl.BlockSpec((ROWB, D), lambda i: (i, 0)),
        out_shape=jax.ShapeDtypeStruct((N, D), jnp.float32),
    )(aggp, degp, xr, W_l)


def kernel(x, edge_index, W1_l, b1, W1_r, W2_l, b2, W2_r):
    src = edge_index[0].astype(jnp.int32)
    dst = edge_index[1].astype(jnp.int32)
    npadedges = EPAD - E
    srcp = jnp.concatenate([src, jnp.zeros((npadedges,), jnp.int32)])
    dstp = jnp.concatenate([dst, jnp.full((npadedges,), PAD_DST, jnp.int32)])
    idx_all = jnp.stack(
        [srcp.reshape(NSUB * NCHUNK, CHUNK),
         dstp.reshape(NSUB * NCHUNK, CHUNK)], axis=1,
    ).reshape(NSUB * NCHUNK * 2, CHUNK)
    zrow = jnp.zeros((CHUNK, DH), jnp.float32)
    zdeg = jnp.zeros((CHUNK, 16), jnp.float32)
    ones = jnp.ones((CHUNK, 16), jnp.float32)

    x0 = x[:, :DH]
    x1 = x[:, DH:]
    xr1 = _tc_matmul(x, W1_r, b1.reshape(1, D))
    aggp, degp = _sc_pass(x0, x1, idx_all, zrow, zdeg, ones)
    h = _tc_combine(aggp, degp, xr1, W1_l, relu=True)
    xr2 = _tc_matmul(h, W2_r, b2.reshape(1, D))
    aggp2, degp2 = _sc_pass(h[:, :DH], h[:, DH:], idx_all, zrow, zdeg, ones)
    out = _tc_combine(aggp2, degp2, xr2, W2_l, relu=False)
    return out


# speedup vs baseline: 1.0192x; 1.0192x over previous
import functools

import jax
import jax.numpy as jnp
from jax import lax
from jax.experimental import pallas as pl
from jax.experimental.pallas import tpu as pltpu
from jax.experimental.pallas import tpu_sc as plsc

N = 10000
D = 128
DH = D // 2
E = 320000

NPAD = 10240
PAD_DST = 10008
NCORES = 2
NSUB = 16
CHUNK = 128
NCHUNK = 160
BCHUNKS = 10
EPT = CHUNK * NCHUNK
EPAD = EPT * NSUB
RPT = NPAD // NSUB
RCHUNKS = RPT // CHUNK


def _sc_body(x0_hbm, x1_hbm, idx_hbm, zrow_hbm, zdeg_hbm, ones_hbm,
             agg_out, deg_out, ib_v, rows_a, rows_b, rows_c, rows_d, rows_e,
             rows_f, ones_v, zbuf_v, zdbuf_v, acc_sh, deg_sh,
             sem_g0, sem_g1, sem_g2, sem_s0, sem_s1, sem_d0, sem_d1):
    c = lax.axis_index("c")
    s = lax.axis_index("s")
    r0 = s * RPT

    pltpu.sync_copy(zrow_hbm, zbuf_v)
    pltpu.sync_copy(zdeg_hbm, zdbuf_v)
    pltpu.sync_copy(ones_hbm, ones_v)
    for k in range(RCHUNKS):
        pltpu.sync_copy(zbuf_v, acc_sh.at[pl.ds(r0 + k * CHUNK, CHUNK)])
        pltpu.sync_copy(zdbuf_v, deg_sh.at[pl.ds(r0 + k * CHUNK, CHUNK)])
    plsc.subcore_barrier()

    rows = (rows_a, rows_b, rows_c, rows_d, rows_e, rows_f)
    sem_g = (sem_g0, sem_g1, sem_g2)
    sem_s = (sem_s0, sem_s1)
    sem_d = (sem_d0, sem_d1)
    ibase = s * (NCHUNK * 2)

    @pl.loop(0, NCHUNK // BCHUNKS)
    def _(bi):
        pltpu.sync_copy(idx_hbm.at[pl.ds(ibase + bi * (2 * BCHUNKS),
                                         2 * BCHUNKS)], ib_v)

        def issue_gather(j):
            r = rows[j % 6]
            g = sem_g[j % 3]

            @pl.when(c == 0)
            def _():
                pltpu.async_copy(x0_hbm.at[ib_v.at[2 * j]], r, g)

            @pl.when(c == 1)
            def _():
                pltpu.async_copy(x1_hbm.at[ib_v.at[2 * j]], r, g)

        def wait_gather(j):
            pltpu.make_async_copy(x0_hbm.at[ib_v.at[2 * j]], rows[j % 6],
                                  sem_g[j % 3]).wait()

        def issue_scatter(j):
            pltpu.async_copy(rows[j % 6], acc_sh.at[ib_v.at[2 * j + 1]],
                             sem_s[j % 2], add=True)

            @pl.when(c == j % 2)
            def _():
                pltpu.async_copy(ones_v, deg_sh.at[ib_v.at[2 * j + 1]],
                                 sem_d[j % 2], add=True)

        def wait_scatter(j):
            pltpu.make_async_copy(rows[j % 6], acc_sh.at[ib_v.at[2 * j + 1]],
                                  sem_s[j % 2]).wait()

            @pl.when(c == j % 2)
            def _():
                pltpu.make_async_copy(ones_v, deg_sh.at[ib_v.at[2 * j + 1]],
                                      sem_d[j % 2]).wait()

        issue_gather(0)
        issue_gather(1)
        issue_gather(2)
        for j in range(BCHUNKS):
            wait_gather(j)
            if j >= 2:
                wait_scatter(j - 2)
            issue_scatter(j)
            if j + 3 < BCHUNKS:
                issue_gather(j + 3)
        wait_scatter(BCHUNKS - 2)
        wait_scatter(BCHUNKS - 1)

    plsc.subcore_barrier()
    out0 = c * NPAD + r0
    for k in range(3):
        pltpu.async_copy(acc_sh.at[pl.ds(r0 + k * CHUNK, CHUNK)],
                         rows[k % 3], sem_g[k % 3])
    for k in range(RCHUNKS):
        pltpu.make_async_copy(acc_sh.at[pl.ds(r0 + k * CHUNK, CHUNK)],
                              rows[k % 3], sem_g[k % 3]).wait()
        pltpu.sync_copy(rows[k % 3],
                        agg_out.at[pl.ds(out0 + k * CHUNK, CHUNK)])
        if k + 3 < RCHUNKS:
            pltpu.async_copy(acc_sh.at[pl.ds(r0 + (k + 3) * CHUNK, CHUNK)],
                             rows[k % 3], sem_g[k % 3])
        pltpu.sync_copy(deg_sh.at[pl.ds(r0 + k * CHUNK, CHUNK)], zdbuf_v)
        pltpu.sync_copy(zdbuf_v, deg_out.at[pl.ds(out0 + k * CHUNK, CHUNK)])


@functools.cache
def _sc_pass_kernel():
    mesh = plsc.VectorSubcoreMesh(core_axis_name="c", subcore_axis_name="s")
    return pl.kernel(
        _sc_body,
        compiler_params=pltpu.CompilerParams(use_tc_tiling_on_sc=False),
        out_type=[
            jax.ShapeDtypeStruct((NCORES * NPAD, DH), jnp.float32),
            jax.ShapeDtypeStruct((NCORES * NPAD, 16), jnp.float32),
        ],
        mesh=mesh,
        scratch_types=[
            pltpu.VMEM((2 * BCHUNKS, CHUNK), jnp.int32),
            pltpu.VMEM((CHUNK, DH), jnp.float32),
            pltpu.VMEM((CHUNK, DH), jnp.float32),
            pltpu.VMEM((CHUNK, DH), jnp.float32),
            pltpu.VMEM((CHUNK, DH), jnp.float32),
            pltpu.VMEM((CHUNK, DH), jnp.float32),
            pltpu.VMEM((CHUNK, DH), jnp.float32),
            pltpu.VMEM((CHUNK, 16), jnp.float32),
            pltpu.VMEM((CHUNK, DH), jnp.float32),
            pltpu.VMEM((CHUNK, 16), jnp.float32),
            pltpu.VMEM_SHARED((NPAD, DH), jnp.float32),
            pltpu.VMEM_SHARED((NPAD, 16), jnp.float32),
            pltpu.SemaphoreType.DMA,
            pltpu.SemaphoreType.DMA,
            pltpu.SemaphoreType.DMA,
            pltpu.SemaphoreType.DMA,
            pltpu.SemaphoreType.DMA,
            pltpu.SemaphoreType.DMA,
            pltpu.SemaphoreType.DMA,
        ],
    )


def _sc_pass(x0, x1, idx_all, zrow, zdeg, ones):
    aggp, degp = _sc_pass_kernel()(x0, x1, idx_all, zrow, zdeg, ones)
    return aggp.reshape(NCORES, NPAD, DH), degp.reshape(NCORES, NPAD, 16)


ROWB = 2000


def _tc_matmul_body(x_ref, w_ref, b_ref, o_ref):
    o_ref[...] = lax.dot_general(
        x_ref[...], w_ref[...], (((1,), (1,)), ((), ())),
        preferred_element_type=jnp.float32) + b_ref[...]


def _tc_matmul(xin, W, b):
    return pl.pallas_call(
        _tc_matmul_body,
        grid=(N // ROWB,),
        in_specs=[
            pl.BlockSpec((ROWB, D), lambda i: (i, 0)),
            pl.BlockSpec((D, D), lambda i: (0, 0)),
            pl.BlockSpec((1, D), lambda i: (0, 0)),
        ],
        out_specs=pl.BlockSpec((ROWB, D), lambda i: (i, 0)),
        out_shape=jax.ShapeDtypeStruct((N, D), jnp.float32),
    )(xin, W, b)


def _tc_combine_body(relu, agg_ref, deg_ref, xr_ref, wl_ref, o_ref):
    a = jnp.concatenate([agg_ref[0], agg_ref[1]], axis=1)
    dg = deg_ref[0][:, 0:1] + deg_ref[1][:, 0:1]
    a = a / jnp.maximum(dg, 1.0)
    acc = lax.dot_general(a, wl_ref[...], (((1,), (1,)), ((), ())),
                          preferred_element_type=jnp.float32)
    acc = acc + xr_ref[...]
    if relu:
        acc = jnp.maximum(acc, 0.0)
    o_ref[...] = acc


def _tc_combine(aggp, degp, xr, W_l, relu):
    return pl.pallas_call(
        functools.partial(_tc_combine_body, relu),
        grid=(N // ROWB,),
        in_specs=[
            pl.BlockSpec((NCORES, ROWB, DH), lambda i: (0, i, 0)),
            pl.BlockSpec((NCORES, ROWB, 16), lambda i: (0, i, 0)),
            pl.BlockSpec((ROWB, D), lambda i: (i, 0)),
            pl.BlockSpec((D, D), lambda i: (0, 0)),
        ],
        out_specs=pl.BlockSpec((ROWB, D), lambda i: (i, 0)),
        out_shape=jax.ShapeDtypeStruct((N, D), jnp.float32),
    )(aggp, degp, xr, W_l)


def kernel(x, edge_index, W1_l, b1, W1_r, W2_l, b2, W2_r):
    src = edge_index[0].astype(jnp.int32)
    dst = edge_index[1].astype(jnp.int32)
    npadedges = EPAD - E
    srcp = jnp.concatenate([src, jnp.zeros((npadedges,), jnp.int32)])
    dstp = jnp.concatenate([dst, jnp.full((npadedges,), PAD_DST, jnp.int32)])
    idx_all = jnp.stack(
        [srcp.reshape(NSUB * NCHUNK, CHUNK),
         dstp.reshape(NSUB * NCHUNK, CHUNK)], axis=1,
    ).reshape(NSUB * NCHUNK * 2, CHUNK)
    zrow = jnp.zeros((CHUNK, DH), jnp.float32)
    zdeg = jnp.zeros((CHUNK, 16), jnp.float32)
    ones = jnp.ones((CHUNK, 16), jnp.float32)

    x0 = x[:, :DH]
    x1 = x[:, DH:]
    xr1 = _tc_matmul(x, W1_r, b1.reshape(1, D))
    aggp, degp = _sc_pass(x0, x1, idx_all, zrow, zdeg, ones)
    h = _tc_combine(aggp, degp, xr1, W1_l, relu=True)
    xr2 = _tc_matmul(h, W2_r, b2.reshape(1, D))
    aggp2, degp2 = _sc_pass(h[:, :DH], h[:, DH:], idx_all, zrow, zdeg, ones)
    out = _tc_combine(aggp2, degp2, xr2, W2_l, relu=False)
    return out
